# Initial kernel scaffold; baseline (speedup 1.0000x reference)
import jax
import jax.numpy as jnp
from jax.experimental import pallas as pl


def kernel(x, lut):
    raise NotImplementedError("write your pallas kernel here")


# speedup vs baseline: 1.2917x; 1.2917x over previous
import functools
import math

import jax
import jax.numpy as jnp
import numpy as np
from jax import lax
from jax.experimental import pallas as pl
from jax.experimental.pallas import tpu as pltpu
from jax.experimental.pallas import tpu_sc as plsc

_D = 32
_SCALE = np.float32(math.sqrt(_D))
_NC = 2
_NS = 16
_NW = _NC * _NS

_CHUNK = 1024


@functools.lru_cache(maxsize=None)
def _make_kernel(B):
    assert B % (_NW * _CHUNK) == 0
    bpw = B // _NW
    nch = bpw // _CHUNK
    mesh = plsc.VectorSubcoreMesh(core_axis_name="c", subcore_axis_name="s")

    @functools.partial(
        pl.kernel,
        mesh=mesh,
        out_type=jax.ShapeDtypeStruct((B, _D), jnp.float32),
        scratch_types=[
            pltpu.VMEM((_CHUNK,), jnp.int32),
            pltpu.VMEM((_CHUNK, _D), jnp.float32),
            pltpu.SemaphoreType.DMA,
        ],
        compiler_params=pltpu.CompilerParams(use_tc_tiling_on_sc=False),
    )
    def emb(x_hbm, lut_hbm, out_hbm, idx_v, rows_v, sem):
        wid = lax.axis_index("s") * _NC + lax.axis_index("c")
        base = wid * bpw

        def chunk_body(ci, carry):
            off = base + ci * _CHUNK
            pltpu.sync_copy(x_hbm.at[pl.ds(off, _CHUNK)], idx_v)
            pltpu.async_copy(lut_hbm.at[idx_v], rows_v, sem).wait()

            def scale_body(j, c):
                rows_v[j, pl.ds(0, 16)] = rows_v[j, pl.ds(0, 16)] * _SCALE
                rows_v[j, pl.ds(16, 16)] = rows_v[j, pl.ds(16, 16)] * _SCALE
                return c

            lax.fori_loop(0, _CHUNK, scale_body, 0)
            pltpu.sync_copy(rows_v, out_hbm.at[pl.ds(off, _CHUNK)])
            return carry

        lax.fori_loop(0, nch, chunk_body, 0)

    return emb


def kernel(x, lut):
    B = x.shape[0] * x.shape[1]
    flat = x.reshape((B,)).astype(jnp.int32)
    out = _make_kernel(B)(flat, lut)
    return out.reshape(x.shape + (_D,))


# speedup vs baseline: 1.4766x; 1.1431x over previous
import functools
import math

import jax
import jax.numpy as jnp
import numpy as np
from jax import lax
from jax.experimental import pallas as pl
from jax.experimental.pallas import tpu as pltpu
from jax.experimental.pallas import tpu_sc as plsc

_D = 32
_SCALE = np.float32(math.sqrt(_D))
_NC = 2
_NS = 16
_NW = _NC * _NS

_CHUNK = 1024
_NBUF = 3
_UNROLL = 8


@functools.lru_cache(maxsize=None)
def _make_kernel(B):
    assert B % (_NW * _CHUNK) == 0
    bpw = B // _NW
    nch = bpw // _CHUNK
    mesh = plsc.VectorSubcoreMesh(core_axis_name="c", subcore_axis_name="s")

    @functools.partial(
        pl.kernel,
        mesh=mesh,
        out_type=jax.ShapeDtypeStruct((B, _D), jnp.float32),
        scratch_types=[
            pltpu.VMEM((bpw,), jnp.int32),
            pltpu.VMEM((_NBUF, _CHUNK, _D), jnp.float32),
            [pltpu.SemaphoreType.DMA] * _NBUF,
            [pltpu.SemaphoreType.DMA] * _NBUF,
        ],
        compiler_params=pltpu.CompilerParams(use_tc_tiling_on_sc=False),
    )
    def emb(x_hbm, lut_hbm, out_hbm, idx_v, rows_v, gsems, ssems):
        wid = lax.axis_index("s") * _NC + lax.axis_index("c")
        base = wid * bpw
        pltpu.sync_copy(x_hbm.at[pl.ds(base, bpw)], idx_v)

        def start_gather(ci):
            b = ci % _NBUF
            return pltpu.async_copy(
                lut_hbm.at[idx_v.at[pl.ds(ci * _CHUNK, _CHUNK)]],
                rows_v.at[b],
                gsems[b],
            )

        def start_store(ci):
            b = ci % _NBUF
            return pltpu.async_copy(
                rows_v.at[b],
                out_hbm.at[pl.ds(base + ci * _CHUNK, _CHUNK)],
                ssems[b],
            )

        def scale(b):
            buf = rows_v.at[b]

            def body(j, c):
                r = j * _UNROLL
                for k in range(_UNROLL):
                    for h in range(2):
                        sl = pl.ds(h * 16, 16)
                        buf[r + k, sl] = buf[r + k, sl] * _SCALE
                return c

            lax.fori_loop(0, _CHUNK // _UNROLL, body, 0)

        gathers = {0: start_gather(0), 1: start_gather(1)}
        stores = {}
        for ci in range(nch):
            b = ci % _NBUF
            gathers.pop(ci).wait()
            scale(b)
            stores[ci] = start_store(ci)
            if ci + 2 < nch:
                if ci - 1 in stores:
                    stores.pop(ci - 1).wait()
                gathers[ci + 2] = start_gather(ci + 2)
        for ci in sorted(stores):
            stores.pop(ci).wait()

    return emb


def kernel(x, lut):
    B = x.shape[0] * x.shape[1]
    flat = x.reshape((B,)).astype(jnp.int32)
    out = _make_kernel(B)(flat, lut)
    return out.reshape(x.shape + (_D,))
